# baseline (speedup 1.0000x reference)
import functools

import jax
import jax.numpy as jnp
from jax import lax
from jax.experimental import pallas as pl
from jax.experimental.pallas import tpu as pltpu
from jax.experimental.pallas import tpu_sc as plsc

BZ = 2048
IN_DIM = 768
HID = 400
ED = 64
NE = 1024

ED_PAD = 128

ENC_BLK = 1024
DEC_BLK = 512


K_AUG = ED + 8


def _encoder_body(x_ref, w1_ref, b1_ref, w2_ref, b2_ref, emb_ref, idx_ref,
                  w1t_ref, w2t_ref, eaug_ref):
    @pl.when(pl.program_id(0) == 0)
    def _init():
        w1t_ref[...] = w1_ref[...].T
        w2t_ref[...] = w2_ref[...].T
        embt = emb_ref[...].T
        e2 = jnp.sum(embt * embt, axis=0, keepdims=True)
        eaug_ref[...] = jnp.concatenate(
            [-2.0 * embt, e2, jnp.zeros((K_AUG - ED - 1, NE), jnp.float32)],
            axis=0)

    x = x_ref[...]
    h1 = jnp.maximum(jnp.dot(x, w1t_ref[...]) + b1_ref[...][None, :], 0.0)
    z = jnp.dot(h1, w2t_ref[...]) + b2_ref[...][None, :]
    zaug = jnp.concatenate(
        [z, jnp.ones((ENC_BLK, 1), jnp.float32),
         jnp.zeros((ENC_BLK, K_AUG - ED - 1), jnp.float32)], axis=1)
    s = jnp.dot(zaug, eaug_ref[...], precision=lax.Precision.HIGHEST)
    m = jnp.min(s, axis=1, keepdims=True)
    ids = lax.broadcasted_iota(jnp.int32, s.shape, 1)
    cand = jnp.where(s == m, ids, NE)
    idx = jnp.min(cand, axis=1)
    idx_ref[...] = idx


def _decoder_body(zq_ref, w3_ref, b3_ref, w4_ref, b4_ref, out_ref,
                  w3t_ref, w4t_ref):
    @pl.when(pl.program_id(0) == 0)
    def _init():
        w3t_ref[...] = jnp.concatenate(
            [w3_ref[...].T, jnp.zeros((ED_PAD - ED, HID), jnp.float32)], axis=0)
        w4t_ref[...] = w4_ref[...].T

    h3 = jnp.maximum(jnp.dot(zq_ref[...], w3t_ref[...]) + b3_ref[...][None, :], 0.0)
    logits = jnp.dot(h3, w4t_ref[...]) + b4_ref[...][None, :]
    out_ref[...] = jax.nn.sigmoid(logits)


_N_ENC = BZ // ENC_BLK


def _encode_argmin(x, w1, b1, w2, b2, emb):
    idx3 = pl.pallas_call(
        _encoder_body,
        grid=(_N_ENC,),
        in_specs=[
            pl.BlockSpec((ENC_BLK, IN_DIM), lambda i: (i, 0)),
            pl.BlockSpec((HID, IN_DIM), lambda i: (0, 0)),
            pl.BlockSpec((HID,), lambda i: (0,)),
            pl.BlockSpec((ED, HID), lambda i: (0, 0)),
            pl.BlockSpec((ED,), lambda i: (0,)),
            pl.BlockSpec((NE, ED), lambda i: (0, 0)),
        ],
        out_specs=pl.BlockSpec((ENC_BLK,), lambda i: (i,)),
        out_shape=jax.ShapeDtypeStruct((BZ,), jnp.int32),
        scratch_shapes=[
            pltpu.VMEM((IN_DIM, HID), jnp.float32),
            pltpu.VMEM((HID, ED), jnp.float32),
            pltpu.VMEM((K_AUG, NE), jnp.float32),
        ],
    )(x, w1, b1, w2, b2, emb)
    return idx3


def _decode(zq, w3, b3, w4, b4):
    return pl.pallas_call(
        _decoder_body,
        grid=(BZ // DEC_BLK,),
        in_specs=[
            pl.BlockSpec((DEC_BLK, ED_PAD), lambda i: (i, 0)),
            pl.BlockSpec((HID, ED), lambda i: (0, 0)),
            pl.BlockSpec((HID,), lambda i: (0,)),
            pl.BlockSpec((IN_DIM, HID), lambda i: (0, 0)),
            pl.BlockSpec((IN_DIM,), lambda i: (0,)),
        ],
        out_specs=pl.BlockSpec((DEC_BLK, IN_DIM), lambda i: (i, 0)),
        out_shape=jax.ShapeDtypeStruct((BZ, IN_DIM), jnp.float32),
        scratch_shapes=[
            pltpu.VMEM((ED_PAD, HID), jnp.float32),
            pltpu.VMEM((HID, IN_DIM), jnp.float32),
        ],
    )(zq, w3, b3, w4, b4)


@functools.cache
def _make_sc_gather():
    info = plsc.get_sparse_core_info()
    nc, ns = info.num_cores, info.num_subcores
    nw = nc * ns
    b_per_w = BZ // nw
    mesh = plsc.VectorSubcoreMesh(core_axis_name="c", subcore_axis_name="s")

    @functools.partial(
        pl.kernel,
        mesh=mesh,
        out_type=jax.ShapeDtypeStruct((BZ, ED_PAD), jnp.float32),
        scratch_types=[
            pltpu.VMEM((b_per_w,), jnp.int32),
            pltpu.VMEM((b_per_w, ED_PAD), jnp.float32),
            pltpu.VMEM_SHARED((NE, ED_PAD), jnp.float32),
            pltpu.SemaphoreType.DMA,
        ],
    )
    def sc_gather(emb_hbm, idx_hbm, out_hbm, idx_v, rows_v, emb_sh, sem):
        wid = lax.axis_index("s") * nc + lax.axis_index("c")
        base = wid * b_per_w
        pltpu.sync_copy(idx_hbm.at[pl.ds(base, b_per_w)], idx_v)
        @pl.when(lax.axis_index("s") == 0)
        def _():
            pltpu.sync_copy(emb_hbm, emb_sh)
        plsc.subcore_barrier()
        pltpu.async_copy(emb_sh.at[idx_v], rows_v, sem).wait()
        pltpu.sync_copy(rows_v, out_hbm.at[pl.ds(base, b_per_w)])

    return sc_gather


def kernel(x, fc1_w, fc1_b, fc2_w, fc2_b, fc3_w, fc3_b, fc4_w, fc4_b, emb):
    idx = _encode_argmin(x, fc1_w, fc1_b, fc2_w, fc2_b, emb)
    emb_pad = jnp.pad(emb, ((0, 0), (0, ED_PAD - ED)))
    zq_pad = _make_sc_gather()(emb_pad, idx)
    return _decode(zq_pad, fc3_w, fc3_b, fc4_w, fc4_b)


# speedup vs baseline: 1.0110x; 1.0110x over previous
import functools

import jax
import jax.numpy as jnp
from jax import lax
from jax.experimental import pallas as pl
from jax.experimental.pallas import tpu as pltpu
from jax.experimental.pallas import tpu_sc as plsc

BZ = 2048
IN_DIM = 768
HID = 400
ED = 64
NE = 1024

ED_PAD = 128

ENC_BLK = 1024
DEC_BLK = 1024


K_AUG = ED + 8


def _encoder_body(x_ref, w1_ref, b1_ref, w2_ref, b2_ref, emb_ref, idx_ref,
                  w1t_ref, w2t_ref, eaug_ref):
    @pl.when(pl.program_id(0) == 0)
    def _init():
        w1t_ref[...] = w1_ref[...].T
        w2t_ref[...] = w2_ref[...].T
        embt = emb_ref[...].T
        e2 = jnp.sum(embt * embt, axis=0, keepdims=True)
        eaug_ref[...] = jnp.concatenate(
            [-2.0 * embt, e2, jnp.zeros((K_AUG - ED - 1, NE), jnp.float32)],
            axis=0)

    x = x_ref[...]
    h1 = jnp.maximum(jnp.dot(x, w1t_ref[...]) + b1_ref[...][None, :], 0.0)
    z = jnp.dot(h1, w2t_ref[...]) + b2_ref[...][None, :]
    zaug = jnp.concatenate(
        [z, jnp.ones((ENC_BLK, 1), jnp.float32),
         jnp.zeros((ENC_BLK, K_AUG - ED - 1), jnp.float32)], axis=1)
    s = jnp.dot(zaug, eaug_ref[...], precision=lax.Precision.HIGHEST)
    m = jnp.min(s, axis=1, keepdims=True)
    ids = lax.broadcasted_iota(jnp.int32, s.shape, 1)
    cand = jnp.where(s == m, ids, NE)
    idx = jnp.min(cand, axis=1)
    idx_ref[...] = idx


def _decoder_body(zq_ref, w3_ref, b3_ref, w4_ref, b4_ref, out_ref,
                  w3t_ref, w4t_ref):
    @pl.when(pl.program_id(0) == 0)
    def _init():
        w3t_ref[...] = jnp.concatenate(
            [w3_ref[...].T, jnp.zeros((ED_PAD - ED, HID), jnp.float32)], axis=0)
        w4t_ref[...] = w4_ref[...].T

    h3 = jnp.maximum(jnp.dot(zq_ref[...], w3t_ref[...]) + b3_ref[...][None, :], 0.0)
    logits = jnp.dot(h3, w4t_ref[...]) + b4_ref[...][None, :]
    out_ref[...] = jax.nn.sigmoid(logits)


_N_ENC = BZ // ENC_BLK


def _encode_argmin(x, w1, b1, w2, b2, emb):
    idx3 = pl.pallas_call(
        _encoder_body,
        grid=(_N_ENC,),
        in_specs=[
            pl.BlockSpec((ENC_BLK, IN_DIM), lambda i: (i, 0)),
            pl.BlockSpec((HID, IN_DIM), lambda i: (0, 0)),
            pl.BlockSpec((HID,), lambda i: (0,)),
            pl.BlockSpec((ED, HID), lambda i: (0, 0)),
            pl.BlockSpec((ED,), lambda i: (0,)),
            pl.BlockSpec((NE, ED), lambda i: (0, 0)),
        ],
        out_specs=pl.BlockSpec((ENC_BLK,), lambda i: (i,)),
        out_shape=jax.ShapeDtypeStruct((BZ,), jnp.int32),
        scratch_shapes=[
            pltpu.VMEM((IN_DIM, HID), jnp.float32),
            pltpu.VMEM((HID, ED), jnp.float32),
            pltpu.VMEM((K_AUG, NE), jnp.float32),
        ],
    )(x, w1, b1, w2, b2, emb)
    return idx3


def _decode(zq, w3, b3, w4, b4):
    return pl.pallas_call(
        _decoder_body,
        grid=(BZ // DEC_BLK,),
        in_specs=[
            pl.BlockSpec((DEC_BLK, ED_PAD), lambda i: (i, 0)),
            pl.BlockSpec((HID, ED), lambda i: (0, 0)),
            pl.BlockSpec((HID,), lambda i: (0,)),
            pl.BlockSpec((IN_DIM, HID), lambda i: (0, 0)),
            pl.BlockSpec((IN_DIM,), lambda i: (0,)),
        ],
        out_specs=pl.BlockSpec((DEC_BLK, IN_DIM), lambda i: (i, 0)),
        out_shape=jax.ShapeDtypeStruct((BZ, IN_DIM), jnp.float32),
        scratch_shapes=[
            pltpu.VMEM((ED_PAD, HID), jnp.float32),
            pltpu.VMEM((HID, IN_DIM), jnp.float32),
        ],
    )(zq, w3, b3, w4, b4)


@functools.cache
def _make_sc_gather():
    info = plsc.get_sparse_core_info()
    nc, ns = info.num_cores, info.num_subcores
    nw = nc * ns
    b_per_w = BZ // nw
    mesh = plsc.VectorSubcoreMesh(core_axis_name="c", subcore_axis_name="s")

    @functools.partial(
        pl.kernel,
        mesh=mesh,
        out_type=jax.ShapeDtypeStruct((BZ, ED_PAD), jnp.float32),
        scratch_types=[
            pltpu.VMEM((b_per_w,), jnp.int32),
            pltpu.VMEM((b_per_w, ED_PAD), jnp.float32),
            pltpu.VMEM_SHARED((NE, ED_PAD), jnp.float32),
            pltpu.SemaphoreType.DMA,
        ],
    )
    def sc_gather(emb_hbm, idx_hbm, out_hbm, idx_v, rows_v, emb_sh, sem):
        wid = lax.axis_index("s") * nc + lax.axis_index("c")
        base = wid * b_per_w
        pltpu.sync_copy(idx_hbm.at[pl.ds(base, b_per_w)], idx_v)
        @pl.when(lax.axis_index("s") == 0)
        def _():
            pltpu.sync_copy(emb_hbm, emb_sh)
        plsc.subcore_barrier()
        pltpu.async_copy(emb_sh.at[idx_v], rows_v, sem).wait()
        pltpu.sync_copy(rows_v, out_hbm.at[pl.ds(base, b_per_w)])

    return sc_gather


def kernel(x, fc1_w, fc1_b, fc2_w, fc2_b, fc3_w, fc3_b, fc4_w, fc4_b, emb):
    idx = _encode_argmin(x, fc1_w, fc1_b, fc2_w, fc2_b, emb)
    emb_pad = jnp.pad(emb, ((0, 0), (0, ED_PAD - ED)))
    zq_pad = _make_sc_gather()(emb_pad, idx)
    return _decode(zq_pad, fc3_w, fc3_b, fc4_w, fc4_b)
